# baseline (speedup 1.0000x reference)
import functools

import jax
import jax.numpy as jnp
from jax import lax
from jax.experimental import pallas as pl
from jax.experimental.pallas import tpu as pltpu
from jax.experimental.pallas import tpu_sc as plsc

N = 10000
E = 320000
H = 128
EIN = 16
DEPTH = 5
G = 64
HID = 64
BN_EPS = 1e-5

NW = 32
EPAD = 327680
NSPLIT = 2
EPH = EPAD // NSPLIT
EWH = EPH // NW
IDXROWS = EPAD // 128
IRH = EWH // 128
CA = 512
NCH_A = EWH // CA
DA = 2
CC = 128
NCH_C = EWH // CC
NP = 10240
NPR = NP // 16

_mesh = plsc.VectorSubcoreMesh(core_axis_name="c", subcore_axis_name="s")


def _sc_gather_body(x_hbm, p2_hbm, srcm_hbm, dstm_hbm, xs_hbm, pd_hbm,
                    idx_s, idx_d, xs_v, pd_v,
                    sem_i, sem_g0, sem_g1, sem_w0, sem_w1):
    cid = lax.axis_index("c")
    sid = lax.axis_index("s")
    wid = sid * 2 + cid
    sem_g = (sem_g0, sem_g1)
    sem_w = (sem_w0, sem_w1)

    i0 = pltpu.async_copy(srcm_hbm.at[wid], idx_s, sem_i)
    i1 = pltpu.async_copy(dstm_hbm.at[wid], idx_d, sem_i)
    i0.wait()
    i1.wait()

    LAG = DA - 1
    gd = [None] * NCH_A
    wd = [None] * NCH_A
    for k in range(NCH_A + LAG):
        if k < NCH_A:
            b = k % DA
            if k >= DA:
                for d in wd[k - DA]:
                    d.wait()
            gd[k] = [
                pltpu.async_copy(x_hbm.at[idx_s.at[pl.ds(k * CA, CA)]],
                                 xs_v.at[b], sem_g[b]),
                pltpu.async_copy(p2_hbm.at[idx_d.at[pl.ds(k * CA, CA)]],
                                 pd_v.at[b], sem_g[b]),
            ]
        if k >= LAG:
            kp = k - LAG
            bp = kp % DA
            for d in gd[kp]:
                d.wait()
            b0 = wid * EWH + kp * CA
            wd[kp] = [
                pltpu.async_copy(xs_v.at[bp], xs_hbm.at[pl.ds(b0, CA)],
                                 sem_w[bp]),
                pltpu.async_copy(pd_v.at[bp], pd_hbm.at[pl.ds(b0, CA)],
                                 sem_w[bp]),
            ]
    for k in range(NCH_A - DA, NCH_A):
        for d in wd[k]:
            d.wait()


_sc_gather = functools.partial(
    pl.kernel,
    out_type=(
        jax.ShapeDtypeStruct((EPH, H // 2), jnp.int32),
        jax.ShapeDtypeStruct((EPH, HID // 2), jnp.int32),
    ),
    mesh=_mesh,
    scratch_types=[
        pltpu.VMEM((EWH,), jnp.int32),
        pltpu.VMEM((EWH,), jnp.int32),
        pltpu.VMEM((DA, CA, H // 2), jnp.int32),
        pltpu.VMEM((DA, CA, HID // 2), jnp.int32),
        pltpu.SemaphoreType.DMA,
        pltpu.SemaphoreType.DMA,
        pltpu.SemaphoreType.DMA,
        pltpu.SemaphoreType.DMA,
        pltpu.SemaphoreType.DMA,
    ],
    compiler_params=pltpu.CompilerParams(use_tc_tiling_on_sc=False),
)(_sc_gather_body)


def _sc_aggr_body(msg_hbm, dstm_hbm, zeros_hbm, aggr_hbm,
                  idx_d, msg_v, aggr_sh, sem_m0, sem_m1, sem_i0, sem_i1):
    cid = lax.axis_index("c")
    sid = lax.axis_index("s")
    wid = sid * 2 + cid
    sem_m = (sem_m0, sem_m1)
    sem_i = (sem_i0, sem_i1)

    zd = pltpu.async_copy(zeros_hbm.at[pl.ds(sid * NPR, NPR)],
                          aggr_sh.at[pl.ds(sid * NPR, NPR)], sem_i[0])
    pltpu.sync_copy(dstm_hbm.at[wid], idx_d)
    zd.wait()
    plsc.subcore_barrier()

    md = [None] * NCH_C
    for k in range(NCH_C + 1):
        if k < NCH_C:
            b = k % 2
            b0 = wid * EWH + k * CC
            md[k] = pltpu.async_copy(msg_hbm.at[pl.ds(b0, CC)],
                                     msg_v.at[b], sem_m[b])
        if k >= 1:
            kp = k - 1
            bp = kp % 2
            md[kp].wait()
            pltpu.sync_copy(msg_v.at[bp],
                            aggr_sh.at[idx_d.at[pl.ds(kp * CC, CC)]],
                            add=True)

    plsc.subcore_barrier()
    pltpu.sync_copy(aggr_sh.at[pl.ds(sid * NPR, NPR)],
                    aggr_hbm.at[cid, pl.ds(sid * NPR, NPR)])


_sc_aggr = functools.partial(
    pl.kernel,
    out_type=jax.ShapeDtypeStruct((2, NP, H), jnp.float32),
    mesh=_mesh,
    scratch_types=[
        pltpu.VMEM((EWH,), jnp.int32),
        pltpu.VMEM((2, CC, H), jnp.float32),
        pltpu.VMEM_SHARED((NP, H), jnp.float32),
        pltpu.SemaphoreType.DMA,
        pltpu.SemaphoreType.DMA,
        pltpu.SemaphoreType.DMA,
        pltpu.SemaphoreType.DMA,
    ],
)(_sc_aggr_body)


EB = 2048


def _dot(a, b):
    return jnp.dot(a, b, preferred_element_type=jnp.float32)


def _rtne(v):
    b = lax.bitcast_convert_type(v, jnp.int32)
    r = b + jnp.int32(0x7FFF) + lax.shift_right_logical(b, 16).astype(jnp.int32) % 2
    return r & (-65536)


def _pack(v, f):
    ra = _rtne(v[:, :f])
    rb = _rtne(v[:, f:])
    return ra | lax.shift_right_logical(rb, 16).astype(jnp.int32)


def _unpack(p):
    hi = lax.bitcast_convert_type(p & (-65536), jnp.float32)
    lo = lax.bitcast_convert_type(lax.shift_left(p, 16), jnp.float32)
    return jnp.concatenate([hi, lo], axis=1)


def _tc_edge_body(xs_ref, pd_ref, e_ref, w1a_ref, w1c_ref, w2_ref, b2_ref,
                  eo_ref, msg_ref):
    e = _unpack(e_ref[...])
    xs = _unpack(xs_ref[...])
    h = jnp.maximum(_unpack(pd_ref[...]) + _dot(xs, w1a_ref[...])
                    + _dot(e, w1c_ref[...]), 0.0)
    eo = e + _dot(h, w2_ref[...]) + b2_ref[...]
    eo_ref[...] = _pack(eo, H // 2)
    msg_ref[...] = jnp.maximum(xs + eo, 0.0)


_tc_edge = pl.pallas_call(
    _tc_edge_body,
    grid=(EPH // EB,),
    in_specs=[
        pl.BlockSpec((EB, H // 2), lambda i: (i, 0)),
        pl.BlockSpec((EB, HID // 2), lambda i: (i, 0)),
        pl.BlockSpec((EB, H // 2), lambda i: (i, 0)),
        pl.BlockSpec((H, HID), lambda i: (0, 0)),
        pl.BlockSpec((H, HID), lambda i: (0, 0)),
        pl.BlockSpec((HID, H), lambda i: (0, 0)),
        pl.BlockSpec((1, H), lambda i: (0, 0)),
    ],
    out_specs=[
        pl.BlockSpec((EB, H // 2), lambda i: (i, 0)),
        pl.BlockSpec((EB, H), lambda i: (i, 0)),
    ],
    out_shape=(
        jax.ShapeDtypeStruct((EPH, H // 2), jnp.int32),
        jax.ShapeDtypeStruct((EPH, H), jnp.float32),
    ),
)


def _tc_edge0_body(xs_ref, pd_ref, ea_ref, we_ref, be_ref, w1a_ref, w1c_ref,
                   w2_ref, b2_ref, eo_ref, msg_ref):
    e = _dot(ea_ref[...], we_ref[...]) + be_ref[...]
    xs = _unpack(xs_ref[...])
    h = jnp.maximum(_unpack(pd_ref[...]) + _dot(xs, w1a_ref[...])
                    + _dot(e, w1c_ref[...]), 0.0)
    eo = e + _dot(h, w2_ref[...]) + b2_ref[...]
    eo_ref[...] = _pack(eo, H // 2)
    msg_ref[...] = jnp.maximum(xs + eo, 0.0)


_tc_edge0 = pl.pallas_call(
    _tc_edge0_body,
    grid=(EPH // EB,),
    in_specs=[
        pl.BlockSpec((EB, H // 2), lambda i: (i, 0)),
        pl.BlockSpec((EB, HID // 2), lambda i: (i, 0)),
        pl.BlockSpec((EB, EIN), lambda i: (i, 0)),
        pl.BlockSpec((EIN, H), lambda i: (0, 0)),
        pl.BlockSpec((1, H), lambda i: (0, 0)),
        pl.BlockSpec((H, HID), lambda i: (0, 0)),
        pl.BlockSpec((H, HID), lambda i: (0, 0)),
        pl.BlockSpec((HID, H), lambda i: (0, 0)),
        pl.BlockSpec((1, H), lambda i: (0, 0)),
    ],
    out_specs=[
        pl.BlockSpec((EB, H // 2), lambda i: (i, 0)),
        pl.BlockSpec((EB, H), lambda i: (i, 0)),
    ],
    out_shape=(
        jax.ShapeDtypeStruct((EPH, H // 2), jnp.int32),
        jax.ShapeDtypeStruct((EPH, H), jnp.float32),
    ),
)


def _node_update(x, a0, a1, w1, b1, w2, b2, gam, bet):
    t = x + (a0[0, 0:N, :] + a0[1, 0:N, :]) + (a1[0, 0:N, :] + a1[1, 0:N, :])
    u = _dot(jnp.maximum(_dot(t, w1) + b1, 0.0), w2) + b2
    mean = jnp.mean(u, axis=0, keepdims=True)
    var = jnp.mean((u - mean) * (u - mean), axis=0, keepdims=True)
    xb = (u - mean) * lax.rsqrt(var + BN_EPS) * gam + bet
    return x + jnp.maximum(xb, 0.0)


def _tc_node_body(x_ref, a0_ref, a1_ref, w1_ref, b1_ref, w2_ref, b2_ref,
                  gam_ref, bet_ref, nwb_ref, nb_ref, xo_ref, xp_ref, p2_ref):
    xn = _node_update(x_ref[...], a0_ref[...], a1_ref[...], w1_ref[...],
                      b1_ref[...], w2_ref[...], b2_ref[...], gam_ref[...],
                      bet_ref[...])
    xo_ref[...] = xn
    xp_ref[...] = _pack(xn, H // 2)
    p2_ref[...] = _pack(_dot(xn, nwb_ref[...]) + nb_ref[...], HID // 2)


_tc_node = pl.pallas_call(
    _tc_node_body,
    out_shape=(
        jax.ShapeDtypeStruct((N, H), jnp.float32),
        jax.ShapeDtypeStruct((N, H // 2), jnp.int32),
        jax.ShapeDtypeStruct((N, HID // 2), jnp.int32),
    ),
)


def _tc_last_body(x_ref, a0_ref, a1_ref, w1_ref, b1_ref, w2_ref, b2_ref,
                  gam_ref, bet_ref, batch_ref, row_ref, rob_ref, out_ref):
    xn = _node_update(x_ref[...], a0_ref[...], a1_ref[...], w1_ref[...],
                      b1_ref[...], w2_ref[...], b2_ref[...], gam_ref[...],
                      bet_ref[...])
    oh = (lax.broadcasted_iota(jnp.int32, (G, 1), 0)
          == batch_ref[...]).astype(jnp.float32)
    sums = _dot(oh, xn)
    cnt = jnp.sum(oh, axis=1, keepdims=True)
    g = sums / jnp.maximum(cnt, 1.0)
    out_ref[...] = jnp.maximum(_dot(g, row_ref[...]) + rob_ref[...], 0.0)


_tc_last = pl.pallas_call(
    _tc_last_body,
    out_shape=jax.ShapeDtypeStruct((G, H), jnp.float32),
)


def _tc_prep_body(x_ref, nwb_ref, nb_ref, xp_ref, p2_ref):
    xp_ref[...] = _pack(x_ref[...], H // 2)
    p2_ref[...] = _pack(_dot(x_ref[...], nwb_ref[...]) + nb_ref[...], HID // 2)


_tc_prep = pl.pallas_call(
    _tc_prep_body,
    out_shape=(
        jax.ShapeDtypeStruct((N, H // 2), jnp.int32),
        jax.ShapeDtypeStruct((N, HID // 2), jnp.int32),
    ),
)


def kernel(x, edge_index, edge_attr, batch, e_proj_W, e_proj_b, upd_W1,
           upd_b1, upd_W2, upd_b2, conv_W1, conv_b1, conv_W2, conv_b2,
           bn_gamma, bn_beta, ro_W, ro_b):
    pad = EPAD - E
    src = jnp.concatenate([edge_index[0], jnp.zeros((pad,), jnp.int32)])
    dst = jnp.concatenate([edge_index[1], jnp.full((pad,), N, jnp.int32)])
    srcm = src.reshape(NSPLIT, NW, EWH)
    dstm = dst.reshape(NSPLIT, NW, EWH)
    ea3 = jnp.concatenate([edge_attr, jnp.zeros((pad, EIN), jnp.float32)]
                          ).reshape(NSPLIT, EPH, EIN)
    zeros = jnp.zeros((NP, H), jnp.float32)
    batch_row = batch.reshape(1, N)

    be = e_proj_b.reshape(1, H)
    b2 = [upd_b2[l].reshape(1, H) for l in range(DEPTH)]
    cb1 = [conv_b1[l].reshape(1, H) for l in range(DEPTH)]
    cb2 = [conv_b2[l].reshape(1, H) for l in range(DEPTH)]
    gam = [bn_gamma[l].reshape(1, H) for l in range(DEPTH)]
    bet = [bn_beta[l].reshape(1, H) for l in range(DEPTH)]
    w1a = [upd_W1[l, :H, :] for l in range(DEPTH)]
    w1b = [upd_W1[l, H:2 * H, :] for l in range(DEPTH)]
    w1c = [upd_W1[l, 2 * H:, :] for l in range(DEPTH)]
    nb1 = [upd_b1[l].reshape(1, HID) for l in range(DEPTH)]

    xp, p2 = _tc_prep(x, w1b[0], nb1[0])
    e = [None, None]
    out = None
    for l in range(DEPTH):
        msg = [None, None]
        aggr = [None, None]
        for h in range(NSPLIT):
            xs, pd = _sc_gather(xp, p2, srcm[h], dstm[h])
            if l == 0:
                e[h], msg[h] = _tc_edge0(xs, pd, ea3[h], e_proj_W, be,
                                         w1a[l], w1c[l], upd_W2[l], b2[l])
            else:
                e[h], msg[h] = _tc_edge(xs, pd, e[h], w1a[l], w1c[l],
                                        upd_W2[l], b2[l])
            aggr[h] = _sc_aggr(msg[h], dstm[h], zeros)
        if l < DEPTH - 1:
            x, xp, p2 = _tc_node(x, aggr[0], aggr[1], conv_W1[l], cb1[l],
                                 conv_W2[l], cb2[l], gam[l], bet[l],
                                 w1b[l + 1], nb1[l + 1])
        else:
            out = _tc_last(x, aggr[0], aggr[1], conv_W1[l], cb1[l],
                           conv_W2[l], cb2[l], gam[l], bet[l], batch_row,
                           ro_W, ro_b.reshape(1, H))
    return out


# speedup vs baseline: 1.0049x; 1.0049x over previous
import functools

import jax
import jax.numpy as jnp
from jax import lax
from jax.experimental import pallas as pl
from jax.experimental.pallas import tpu as pltpu
from jax.experimental.pallas import tpu_sc as plsc

N = 10000
E = 320000
H = 128
EIN = 16
DEPTH = 5
G = 64
HID = 64
BN_EPS = 1e-5

NW = 32
EPAD = 327680
NSPLIT = 2
EPH = EPAD // NSPLIT
EWH = EPH // NW
IDXROWS = EPAD // 128
IRH = EWH // 128
CA = 128
NCH_A = EWH // CA
DA = 4
CC = 128
NCH_C = EWH // CC
NP = 10240
NPR = NP // 16

_mesh = plsc.VectorSubcoreMesh(core_axis_name="c", subcore_axis_name="s")


def _sc_gather_body(x_hbm, p2_hbm, srcm_hbm, dstm_hbm, xs_hbm, pd_hbm,
                    idx_s, idx_d, xs_v, pd_v,
                    sem_i, sem_g0, sem_g1, sem_g2, sem_g3,
                    sem_w0, sem_w1, sem_w2, sem_w3):
    cid = lax.axis_index("c")
    sid = lax.axis_index("s")
    wid = sid * 2 + cid
    sem_g = (sem_g0, sem_g1, sem_g2, sem_g3)
    sem_w = (sem_w0, sem_w1, sem_w2, sem_w3)

    i0 = pltpu.async_copy(srcm_hbm.at[wid], idx_s, sem_i)
    i1 = pltpu.async_copy(dstm_hbm.at[wid], idx_d, sem_i)
    i0.wait()
    i1.wait()

    LAG = DA - 1
    gd = [None] * NCH_A
    wd = [None] * NCH_A
    for k in range(NCH_A + LAG):
        if k < NCH_A:
            b = k % DA
            if k >= DA:
                for d in wd[k - DA]:
                    d.wait()
            gd[k] = [
                pltpu.async_copy(x_hbm.at[idx_s.at[pl.ds(k * CA, CA)]],
                                 xs_v.at[b], sem_g[b]),
                pltpu.async_copy(p2_hbm.at[idx_d.at[pl.ds(k * CA, CA)]],
                                 pd_v.at[b], sem_g[b]),
            ]
        if k >= LAG:
            kp = k - LAG
            bp = kp % DA
            for d in gd[kp]:
                d.wait()
            b0 = wid * EWH + kp * CA
            wd[kp] = [
                pltpu.async_copy(xs_v.at[bp], xs_hbm.at[pl.ds(b0, CA)],
                                 sem_w[bp]),
                pltpu.async_copy(pd_v.at[bp], pd_hbm.at[pl.ds(b0, CA)],
                                 sem_w[bp]),
            ]
    for k in range(NCH_A - DA, NCH_A):
        for d in wd[k]:
            d.wait()


_sc_gather = functools.partial(
    pl.kernel,
    out_type=(
        jax.ShapeDtypeStruct((EPH, H // 2), jnp.int32),
        jax.ShapeDtypeStruct((EPH, HID // 2), jnp.int32),
    ),
    mesh=_mesh,
    scratch_types=[
        pltpu.VMEM((EWH,), jnp.int32),
        pltpu.VMEM((EWH,), jnp.int32),
        pltpu.VMEM((DA, CA, H // 2), jnp.int32),
        pltpu.VMEM((DA, CA, HID // 2), jnp.int32),
        pltpu.SemaphoreType.DMA,
        pltpu.SemaphoreType.DMA,
        pltpu.SemaphoreType.DMA,
        pltpu.SemaphoreType.DMA,
        pltpu.SemaphoreType.DMA,
        pltpu.SemaphoreType.DMA,
        pltpu.SemaphoreType.DMA,
        pltpu.SemaphoreType.DMA,
        pltpu.SemaphoreType.DMA,
    ],
    compiler_params=pltpu.CompilerParams(use_tc_tiling_on_sc=False),
)(_sc_gather_body)


def _sc_aggr_body(msg_hbm, dstm_hbm, zeros_hbm, aggr_hbm,
                  idx_d, msg_v, aggr_sh, sem_m0, sem_m1, sem_i0, sem_i1):
    cid = lax.axis_index("c")
    sid = lax.axis_index("s")
    wid = sid * 2 + cid
    sem_m = (sem_m0, sem_m1)
    sem_i = (sem_i0, sem_i1)

    zd = pltpu.async_copy(zeros_hbm.at[pl.ds(sid * NPR, NPR)],
                          aggr_sh.at[pl.ds(sid * NPR, NPR)], sem_i[0])
    pltpu.sync_copy(dstm_hbm.at[wid], idx_d)
    zd.wait()
    plsc.subcore_barrier()

    md = [None] * NCH_C
    for k in range(NCH_C + 1):
        if k < NCH_C:
            b = k % 2
            b0 = wid * EWH + k * CC
            md[k] = pltpu.async_copy(msg_hbm.at[pl.ds(b0, CC)],
                                     msg_v.at[b], sem_m[b])
        if k >= 1:
            kp = k - 1
            bp = kp % 2
            md[kp].wait()
            pltpu.sync_copy(msg_v.at[bp],
                            aggr_sh.at[idx_d.at[pl.ds(kp * CC, CC)]],
                            add=True)

    plsc.subcore_barrier()
    pltpu.sync_copy(aggr_sh.at[pl.ds(sid * NPR, NPR)],
                    aggr_hbm.at[cid, pl.ds(sid * NPR, NPR)])


_sc_aggr = functools.partial(
    pl.kernel,
    out_type=jax.ShapeDtypeStruct((2, NP, H), jnp.float32),
    mesh=_mesh,
    scratch_types=[
        pltpu.VMEM((EWH,), jnp.int32),
        pltpu.VMEM((2, CC, H), jnp.float32),
        pltpu.VMEM_SHARED((NP, H), jnp.float32),
        pltpu.SemaphoreType.DMA,
        pltpu.SemaphoreType.DMA,
        pltpu.SemaphoreType.DMA,
        pltpu.SemaphoreType.DMA,
    ],
)(_sc_aggr_body)


EB = 2048


def _dot(a, b):
    return jnp.dot(a, b, preferred_element_type=jnp.float32)


def _rtne(v):
    b = lax.bitcast_convert_type(v, jnp.int32)
    r = b + jnp.int32(0x7FFF) + lax.shift_right_logical(b, 16).astype(jnp.int32) % 2
    return r & (-65536)


def _pack(v, f):
    ra = _rtne(v[:, :f])
    rb = _rtne(v[:, f:])
    return ra | lax.shift_right_logical(rb, 16).astype(jnp.int32)


def _unpack(p):
    hi = lax.bitcast_convert_type(p & (-65536), jnp.float32)
    lo = lax.bitcast_convert_type(lax.shift_left(p, 16), jnp.float32)
    return jnp.concatenate([hi, lo], axis=1)


def _tc_edge_body(xs_ref, pd_ref, e_ref, w1a_ref, w1c_ref, w2_ref, b2_ref,
                  eo_ref, msg_ref):
    e = _unpack(e_ref[...])
    xs = _unpack(xs_ref[...])
    h = jnp.maximum(_unpack(pd_ref[...]) + _dot(xs, w1a_ref[...])
                    + _dot(e, w1c_ref[...]), 0.0)
    eo = e + _dot(h, w2_ref[...]) + b2_ref[...]
    eo_ref[...] = _pack(eo, H // 2)
    msg_ref[...] = jnp.maximum(xs + eo, 0.0)


_tc_edge = pl.pallas_call(
    _tc_edge_body,
    grid=(EPH // EB,),
    in_specs=[
        pl.BlockSpec((EB, H // 2), lambda i: (i, 0)),
        pl.BlockSpec((EB, HID // 2), lambda i: (i, 0)),
        pl.BlockSpec((EB, H // 2), lambda i: (i, 0)),
        pl.BlockSpec((H, HID), lambda i: (0, 0)),
        pl.BlockSpec((H, HID), lambda i: (0, 0)),
        pl.BlockSpec((HID, H), lambda i: (0, 0)),
        pl.BlockSpec((1, H), lambda i: (0, 0)),
    ],
    out_specs=[
        pl.BlockSpec((EB, H // 2), lambda i: (i, 0)),
        pl.BlockSpec((EB, H), lambda i: (i, 0)),
    ],
    out_shape=(
        jax.ShapeDtypeStruct((EPH, H // 2), jnp.int32),
        jax.ShapeDtypeStruct((EPH, H), jnp.float32),
    ),
)


def _tc_edge0_body(xs_ref, pd_ref, ea_ref, we_ref, be_ref, w1a_ref, w1c_ref,
                   w2_ref, b2_ref, eo_ref, msg_ref):
    e = _dot(ea_ref[...], we_ref[...]) + be_ref[...]
    xs = _unpack(xs_ref[...])
    h = jnp.maximum(_unpack(pd_ref[...]) + _dot(xs, w1a_ref[...])
                    + _dot(e, w1c_ref[...]), 0.0)
    eo = e + _dot(h, w2_ref[...]) + b2_ref[...]
    eo_ref[...] = _pack(eo, H // 2)
    msg_ref[...] = jnp.maximum(xs + eo, 0.0)


_tc_edge0 = pl.pallas_call(
    _tc_edge0_body,
    grid=(EPH // EB,),
    in_specs=[
        pl.BlockSpec((EB, H // 2), lambda i: (i, 0)),
        pl.BlockSpec((EB, HID // 2), lambda i: (i, 0)),
        pl.BlockSpec((EB, EIN), lambda i: (i, 0)),
        pl.BlockSpec((EIN, H), lambda i: (0, 0)),
        pl.BlockSpec((1, H), lambda i: (0, 0)),
        pl.BlockSpec((H, HID), lambda i: (0, 0)),
        pl.BlockSpec((H, HID), lambda i: (0, 0)),
        pl.BlockSpec((HID, H), lambda i: (0, 0)),
        pl.BlockSpec((1, H), lambda i: (0, 0)),
    ],
    out_specs=[
        pl.BlockSpec((EB, H // 2), lambda i: (i, 0)),
        pl.BlockSpec((EB, H), lambda i: (i, 0)),
    ],
    out_shape=(
        jax.ShapeDtypeStruct((EPH, H // 2), jnp.int32),
        jax.ShapeDtypeStruct((EPH, H), jnp.float32),
    ),
)


def _node_update(x, a0, a1, w1, b1, w2, b2, gam, bet):
    t = x + (a0[0, 0:N, :] + a0[1, 0:N, :]) + (a1[0, 0:N, :] + a1[1, 0:N, :])
    u = _dot(jnp.maximum(_dot(t, w1) + b1, 0.0), w2) + b2
    mean = jnp.mean(u, axis=0, keepdims=True)
    var = jnp.mean((u - mean) * (u - mean), axis=0, keepdims=True)
    xb = (u - mean) * lax.rsqrt(var + BN_EPS) * gam + bet
    return x + jnp.maximum(xb, 0.0)


def _tc_node_body(x_ref, a0_ref, a1_ref, w1_ref, b1_ref, w2_ref, b2_ref,
                  gam_ref, bet_ref, nwb_ref, nb_ref, xo_ref, xp_ref, p2_ref):
    xn = _node_update(x_ref[...], a0_ref[...], a1_ref[...], w1_ref[...],
                      b1_ref[...], w2_ref[...], b2_ref[...], gam_ref[...],
                      bet_ref[...])
    xo_ref[...] = xn
    xp_ref[...] = _pack(xn, H // 2)
    p2_ref[...] = _pack(_dot(xn, nwb_ref[...]) + nb_ref[...], HID // 2)


_tc_node = pl.pallas_call(
    _tc_node_body,
    out_shape=(
        jax.ShapeDtypeStruct((N, H), jnp.float32),
        jax.ShapeDtypeStruct((N, H // 2), jnp.int32),
        jax.ShapeDtypeStruct((N, HID // 2), jnp.int32),
    ),
)


def _tc_last_body(x_ref, a0_ref, a1_ref, w1_ref, b1_ref, w2_ref, b2_ref,
                  gam_ref, bet_ref, batch_ref, row_ref, rob_ref, out_ref):
    xn = _node_update(x_ref[...], a0_ref[...], a1_ref[...], w1_ref[...],
                      b1_ref[...], w2_ref[...], b2_ref[...], gam_ref[...],
                      bet_ref[...])
    oh = (lax.broadcasted_iota(jnp.int32, (G, 1), 0)
          == batch_ref[...]).astype(jnp.float32)
    sums = _dot(oh, xn)
    cnt = jnp.sum(oh, axis=1, keepdims=True)
    g = sums / jnp.maximum(cnt, 1.0)
    out_ref[...] = jnp.maximum(_dot(g, row_ref[...]) + rob_ref[...], 0.0)


_tc_last = pl.pallas_call(
    _tc_last_body,
    out_shape=jax.ShapeDtypeStruct((G, H), jnp.float32),
)


def _tc_prep_body(x_ref, nwb_ref, nb_ref, xp_ref, p2_ref):
    xp_ref[...] = _pack(x_ref[...], H // 2)
    p2_ref[...] = _pack(_dot(x_ref[...], nwb_ref[...]) + nb_ref[...], HID // 2)


_tc_prep = pl.pallas_call(
    _tc_prep_body,
    out_shape=(
        jax.ShapeDtypeStruct((N, H // 2), jnp.int32),
        jax.ShapeDtypeStruct((N, HID // 2), jnp.int32),
    ),
)


def kernel(x, edge_index, edge_attr, batch, e_proj_W, e_proj_b, upd_W1,
           upd_b1, upd_W2, upd_b2, conv_W1, conv_b1, conv_W2, conv_b2,
           bn_gamma, bn_beta, ro_W, ro_b):
    pad = EPAD - E
    src = jnp.concatenate([edge_index[0], jnp.zeros((pad,), jnp.int32)])
    dst = jnp.concatenate([edge_index[1], jnp.full((pad,), N, jnp.int32)])
    srcm = src.reshape(NSPLIT, NW, EWH)
    dstm = dst.reshape(NSPLIT, NW, EWH)
    ea3 = jnp.concatenate([edge_attr, jnp.zeros((pad, EIN), jnp.float32)]
                          ).reshape(NSPLIT, EPH, EIN)
    zeros = jnp.zeros((NP, H), jnp.float32)
    batch_row = batch.reshape(1, N)

    be = e_proj_b.reshape(1, H)
    b2 = [upd_b2[l].reshape(1, H) for l in range(DEPTH)]
    cb1 = [conv_b1[l].reshape(1, H) for l in range(DEPTH)]
    cb2 = [conv_b2[l].reshape(1, H) for l in range(DEPTH)]
    gam = [bn_gamma[l].reshape(1, H) for l in range(DEPTH)]
    bet = [bn_beta[l].reshape(1, H) for l in range(DEPTH)]
    w1a = [upd_W1[l, :H, :] for l in range(DEPTH)]
    w1b = [upd_W1[l, H:2 * H, :] for l in range(DEPTH)]
    w1c = [upd_W1[l, 2 * H:, :] for l in range(DEPTH)]
    nb1 = [upd_b1[l].reshape(1, HID) for l in range(DEPTH)]

    xp, p2 = _tc_prep(x, w1b[0], nb1[0])
    e = [None, None]
    out = None
    for l in range(DEPTH):
        msg = [None, None]
        aggr = [None, None]
        for h in range(NSPLIT):
            xs, pd = _sc_gather(xp, p2, srcm[h], dstm[h])
            if l == 0:
                e[h], msg[h] = _tc_edge0(xs, pd, ea3[h], e_proj_W, be,
                                         w1a[l], w1c[l], upd_W2[l], b2[l])
            else:
                e[h], msg[h] = _tc_edge(xs, pd, e[h], w1a[l], w1c[l],
                                        upd_W2[l], b2[l])
            aggr[h] = _sc_aggr(msg[h], dstm[h], zeros)
        if l < DEPTH - 1:
            x, xp, p2 = _tc_node(x, aggr[0], aggr[1], conv_W1[l], cb1[l],
                                 conv_W2[l], cb2[l], gam[l], bet[l],
                                 w1b[l + 1], nb1[l + 1])
        else:
            out = _tc_last(x, aggr[0], aggr[1], conv_W1[l], cb1[l],
                           conv_W2[l], cb2[l], gam[l], bet[l], batch_row,
                           ro_W, ro_b.reshape(1, H))
    return out


# speedup vs baseline: 1.0184x; 1.0135x over previous
import functools

import jax
import jax.numpy as jnp
from jax import lax
from jax.experimental import pallas as pl
from jax.experimental.pallas import tpu as pltpu
from jax.experimental.pallas import tpu_sc as plsc

N = 10000
E = 320000
H = 128
EIN = 16
DEPTH = 5
G = 64
HID = 64
BN_EPS = 1e-5

NW = 32
EPAD = 327680
NSPLIT = 2
EPH = EPAD // NSPLIT
EWH = EPH // NW
IDXROWS = EPAD // 128
IRH = EWH // 128
CA = 128
NCH_A = EWH // CA
DA = 4
CC = 128
NCH_C = EWH // CC
NP = 10240
NPR = NP // 16

_mesh = plsc.VectorSubcoreMesh(core_axis_name="c", subcore_axis_name="s")


def _sc_gather_body(x_hbm, p2_hbm, srcm_hbm, dstm_hbm, xs_hbm, pd_hbm,
                    idx_s, idx_d, xs_v, pd_v,
                    sem_i, sem_g0, sem_g1, sem_g2, sem_g3,
                    sem_w0, sem_w1, sem_w2, sem_w3):
    cid = lax.axis_index("c")
    sid = lax.axis_index("s")
    wid = sid * 2 + cid
    sem_g = (sem_g0, sem_g1, sem_g2, sem_g3)
    sem_w = (sem_w0, sem_w1, sem_w2, sem_w3)

    r0 = wid * IRH
    i0 = pltpu.async_copy(srcm_hbm.at[pl.ds(r0, IRH)], idx_s, sem_i)
    i1 = pltpu.async_copy(dstm_hbm.at[pl.ds(r0, IRH)], idx_d, sem_i)
    i0.wait()
    i1.wait()

    LAG = DA - 1
    gd = [None] * NCH_A
    wd = [None] * NCH_A
    for k in range(NCH_A + LAG):
        if k < NCH_A:
            b = k % DA
            if k >= DA:
                for d in wd[k - DA]:
                    d.wait()
            gd[k] = [
                pltpu.async_copy(x_hbm.at[idx_s.at[k]],
                                 xs_v.at[b], sem_g[b]),
                pltpu.async_copy(p2_hbm.at[idx_d.at[k]],
                                 pd_v.at[b], sem_g[b]),
            ]
        if k >= LAG:
            kp = k - LAG
            bp = kp % DA
            for d in gd[kp]:
                d.wait()
            b0 = wid * EWH + kp * CA
            wd[kp] = [
                pltpu.async_copy(xs_v.at[bp], xs_hbm.at[pl.ds(b0, CA)],
                                 sem_w[bp]),
                pltpu.async_copy(pd_v.at[bp], pd_hbm.at[pl.ds(b0, CA)],
                                 sem_w[bp]),
            ]
    for k in range(NCH_A - DA, NCH_A):
        for d in wd[k]:
            d.wait()


_sc_gather = functools.partial(
    pl.kernel,
    out_type=(
        jax.ShapeDtypeStruct((EPH, H // 2), jnp.int32),
        jax.ShapeDtypeStruct((EPH, HID // 2), jnp.int32),
    ),
    mesh=_mesh,
    scratch_types=[
        pltpu.VMEM((IRH, 128), jnp.int32),
        pltpu.VMEM((IRH, 128), jnp.int32),
        pltpu.VMEM((DA, CA, H // 2), jnp.int32),
        pltpu.VMEM((DA, CA, HID // 2), jnp.int32),
        pltpu.SemaphoreType.DMA,
        pltpu.SemaphoreType.DMA,
        pltpu.SemaphoreType.DMA,
        pltpu.SemaphoreType.DMA,
        pltpu.SemaphoreType.DMA,
        pltpu.SemaphoreType.DMA,
        pltpu.SemaphoreType.DMA,
        pltpu.SemaphoreType.DMA,
        pltpu.SemaphoreType.DMA,
    ],
    compiler_params=pltpu.CompilerParams(use_tc_tiling_on_sc=False),
)(_sc_gather_body)


def _sc_aggr_body(msg_hbm, dstm_hbm, zeros_hbm, aggr_hbm,
                  idx_d, msg_v, aggr_sh, sem_m0, sem_m1, sem_i0, sem_i1):
    cid = lax.axis_index("c")
    sid = lax.axis_index("s")
    wid = sid * 2 + cid
    sem_m = (sem_m0, sem_m1)
    sem_i = (sem_i0, sem_i1)

    zd = pltpu.async_copy(zeros_hbm.at[pl.ds(sid * NPR, NPR)],
                          aggr_sh.at[pl.ds(sid * NPR, NPR)], sem_i[0])
    pltpu.sync_copy(dstm_hbm.at[pl.ds(wid * IRH, IRH)], idx_d)
    zd.wait()
    plsc.subcore_barrier()

    md = [None] * NCH_C
    for k in range(NCH_C + 1):
        if k < NCH_C:
            b = k % 2
            b0 = wid * EWH + k * CC
            md[k] = pltpu.async_copy(msg_hbm.at[pl.ds(b0, CC)],
                                     msg_v.at[b], sem_m[b])
        if k >= 1:
            kp = k - 1
            bp = kp % 2
            md[kp].wait()
            pltpu.sync_copy(msg_v.at[bp], aggr_sh.at[idx_d.at[kp]],
                            add=True)

    plsc.subcore_barrier()
    pltpu.sync_copy(aggr_sh.at[pl.ds(sid * NPR, NPR)],
                    aggr_hbm.at[cid, pl.ds(sid * NPR, NPR)])


_sc_aggr = functools.partial(
    pl.kernel,
    out_type=jax.ShapeDtypeStruct((2, NP, H), jnp.float32),
    mesh=_mesh,
    scratch_types=[
        pltpu.VMEM((IRH, 128), jnp.int32),
        pltpu.VMEM((2, CC, H), jnp.float32),
        pltpu.VMEM_SHARED((NP, H), jnp.float32),
        pltpu.SemaphoreType.DMA,
        pltpu.SemaphoreType.DMA,
        pltpu.SemaphoreType.DMA,
        pltpu.SemaphoreType.DMA,
    ],
)(_sc_aggr_body)


EB = 2048


def _dot(a, b):
    return jnp.dot(a, b, preferred_element_type=jnp.float32)


def _rtne(v):
    b = lax.bitcast_convert_type(v, jnp.int32)
    r = b + jnp.int32(0x7FFF) + lax.shift_right_logical(b, 16).astype(jnp.int32) % 2
    return r & (-65536)


def _pack(v, f):
    ra = _rtne(v[:, :f])
    rb = _rtne(v[:, f:])
    return ra | lax.shift_right_logical(rb, 16).astype(jnp.int32)


def _unpack(p):
    hi = lax.bitcast_convert_type(p & (-65536), jnp.float32)
    lo = lax.bitcast_convert_type(lax.shift_left(p, 16), jnp.float32)
    return jnp.concatenate([hi, lo], axis=1)


def _tc_edge_body(xs_ref, pd_ref, e_ref, w1a_ref, w1c_ref, w2_ref, b2_ref,
                  eo_ref, msg_ref):
    e = _unpack(e_ref[...])
    xs = _unpack(xs_ref[...])
    h = jnp.maximum(_unpack(pd_ref[...]) + _dot(xs, w1a_ref[...])
                    + _dot(e, w1c_ref[...]), 0.0)
    eo = e + _dot(h, w2_ref[...]) + b2_ref[...]
    eo_ref[...] = _pack(eo, H // 2)
    msg_ref[...] = jnp.maximum(xs + eo, 0.0)


_tc_edge = pl.pallas_call(
    _tc_edge_body,
    grid=(EPH // EB,),
    in_specs=[
        pl.BlockSpec((EB, H // 2), lambda i: (i, 0)),
        pl.BlockSpec((EB, HID // 2), lambda i: (i, 0)),
        pl.BlockSpec((EB, H // 2), lambda i: (i, 0)),
        pl.BlockSpec((H, HID), lambda i: (0, 0)),
        pl.BlockSpec((H, HID), lambda i: (0, 0)),
        pl.BlockSpec((HID, H), lambda i: (0, 0)),
        pl.BlockSpec((1, H), lambda i: (0, 0)),
    ],
    out_specs=[
        pl.BlockSpec((EB, H // 2), lambda i: (i, 0)),
        pl.BlockSpec((EB, H), lambda i: (i, 0)),
    ],
    out_shape=(
        jax.ShapeDtypeStruct((EPH, H // 2), jnp.int32),
        jax.ShapeDtypeStruct((EPH, H), jnp.float32),
    ),
)


def _tc_edge0_body(xs_ref, pd_ref, ea_ref, we_ref, be_ref, w1a_ref, w1c_ref,
                   w2_ref, b2_ref, eo_ref, msg_ref):
    e = _dot(ea_ref[...], we_ref[...]) + be_ref[...]
    xs = _unpack(xs_ref[...])
    h = jnp.maximum(_unpack(pd_ref[...]) + _dot(xs, w1a_ref[...])
                    + _dot(e, w1c_ref[...]), 0.0)
    eo = e + _dot(h, w2_ref[...]) + b2_ref[...]
    eo_ref[...] = _pack(eo, H // 2)
    msg_ref[...] = jnp.maximum(xs + eo, 0.0)


_tc_edge0 = pl.pallas_call(
    _tc_edge0_body,
    grid=(EPH // EB,),
    in_specs=[
        pl.BlockSpec((EB, H // 2), lambda i: (i, 0)),
        pl.BlockSpec((EB, HID // 2), lambda i: (i, 0)),
        pl.BlockSpec((EB, EIN), lambda i: (i, 0)),
        pl.BlockSpec((EIN, H), lambda i: (0, 0)),
        pl.BlockSpec((1, H), lambda i: (0, 0)),
        pl.BlockSpec((H, HID), lambda i: (0, 0)),
        pl.BlockSpec((H, HID), lambda i: (0, 0)),
        pl.BlockSpec((HID, H), lambda i: (0, 0)),
        pl.BlockSpec((1, H), lambda i: (0, 0)),
    ],
    out_specs=[
        pl.BlockSpec((EB, H // 2), lambda i: (i, 0)),
        pl.BlockSpec((EB, H), lambda i: (i, 0)),
    ],
    out_shape=(
        jax.ShapeDtypeStruct((EPH, H // 2), jnp.int32),
        jax.ShapeDtypeStruct((EPH, H), jnp.float32),
    ),
)


def _node_update(x, a0, a1, w1, b1, w2, b2, gam, bet):
    t = x + (a0[0, 0:N, :] + a0[1, 0:N, :]) + (a1[0, 0:N, :] + a1[1, 0:N, :])
    u = _dot(jnp.maximum(_dot(t, w1) + b1, 0.0), w2) + b2
    mean = jnp.mean(u, axis=0, keepdims=True)
    var = jnp.mean((u - mean) * (u - mean), axis=0, keepdims=True)
    xb = (u - mean) * lax.rsqrt(var + BN_EPS) * gam + bet
    return x + jnp.maximum(xb, 0.0)


def _tc_node_body(x_ref, a0_ref, a1_ref, w1_ref, b1_ref, w2_ref, b2_ref,
                  gam_ref, bet_ref, nwb_ref, nb_ref, xo_ref, xp_ref, p2_ref):
    xn = _node_update(x_ref[...], a0_ref[...], a1_ref[...], w1_ref[...],
                      b1_ref[...], w2_ref[...], b2_ref[...], gam_ref[...],
                      bet_ref[...])
    xo_ref[...] = xn
    xp_ref[...] = _pack(xn, H // 2)
    p2_ref[...] = _pack(_dot(xn, nwb_ref[...]) + nb_ref[...], HID // 2)


_tc_node = pl.pallas_call(
    _tc_node_body,
    out_shape=(
        jax.ShapeDtypeStruct((N, H), jnp.float32),
        jax.ShapeDtypeStruct((N, H // 2), jnp.int32),
        jax.ShapeDtypeStruct((N, HID // 2), jnp.int32),
    ),
)


def _tc_last_body(x_ref, a0_ref, a1_ref, w1_ref, b1_ref, w2_ref, b2_ref,
                  gam_ref, bet_ref, batch_ref, row_ref, rob_ref, out_ref):
    xn = _node_update(x_ref[...], a0_ref[...], a1_ref[...], w1_ref[...],
                      b1_ref[...], w2_ref[...], b2_ref[...], gam_ref[...],
                      bet_ref[...])
    oh = (lax.broadcasted_iota(jnp.int32, (G, 1), 0)
          == batch_ref[...]).astype(jnp.float32)
    sums = _dot(oh, xn)
    cnt = jnp.sum(oh, axis=1, keepdims=True)
    g = sums / jnp.maximum(cnt, 1.0)
    out_ref[...] = jnp.maximum(_dot(g, row_ref[...]) + rob_ref[...], 0.0)


_tc_last = pl.pallas_call(
    _tc_last_body,
    out_shape=jax.ShapeDtypeStruct((G, H), jnp.float32),
)


def _tc_prep_body(x_ref, nwb_ref, nb_ref, xp_ref, p2_ref):
    xp_ref[...] = _pack(x_ref[...], H // 2)
    p2_ref[...] = _pack(_dot(x_ref[...], nwb_ref[...]) + nb_ref[...], HID // 2)


_tc_prep = pl.pallas_call(
    _tc_prep_body,
    out_shape=(
        jax.ShapeDtypeStruct((N, H // 2), jnp.int32),
        jax.ShapeDtypeStruct((N, HID // 2), jnp.int32),
    ),
)


def kernel(x, edge_index, edge_attr, batch, e_proj_W, e_proj_b, upd_W1,
           upd_b1, upd_W2, upd_b2, conv_W1, conv_b1, conv_W2, conv_b2,
           bn_gamma, bn_beta, ro_W, ro_b):
    pad = EPAD - E
    src = jnp.concatenate([edge_index[0], jnp.zeros((pad,), jnp.int32)])
    dst = jnp.concatenate([edge_index[1], jnp.full((pad,), N, jnp.int32)])
    srcm = src.reshape(NSPLIT, IDXROWS // NSPLIT, 128)
    dstm = dst.reshape(NSPLIT, IDXROWS // NSPLIT, 128)
    ea3 = jnp.concatenate([edge_attr, jnp.zeros((pad, EIN), jnp.float32)]
                          ).reshape(NSPLIT, EPH, EIN)
    zeros = jnp.zeros((NP, H), jnp.float32)
    batch_row = batch.reshape(1, N)

    be = e_proj_b.reshape(1, H)
    b2 = [upd_b2[l].reshape(1, H) for l in range(DEPTH)]
    cb1 = [conv_b1[l].reshape(1, H) for l in range(DEPTH)]
    cb2 = [conv_b2[l].reshape(1, H) for l in range(DEPTH)]
    gam = [bn_gamma[l].reshape(1, H) for l in range(DEPTH)]
    bet = [bn_beta[l].reshape(1, H) for l in range(DEPTH)]
    w1a = [upd_W1[l, :H, :] for l in range(DEPTH)]
    w1b = [upd_W1[l, H:2 * H, :] for l in range(DEPTH)]
    w1c = [upd_W1[l, 2 * H:, :] for l in range(DEPTH)]
    nb1 = [upd_b1[l].reshape(1, HID) for l in range(DEPTH)]

    xp, p2 = _tc_prep(x, w1b[0], nb1[0])
    e = [None, None]
    out = None
    for l in range(DEPTH):
        msg = [None, None]
        aggr = [None, None]
        for h in range(NSPLIT):
            xs, pd = _sc_gather(xp, p2, srcm[h], dstm[h])
            if l == 0:
                e[h], msg[h] = _tc_edge0(xs, pd, ea3[h], e_proj_W, be,
                                         w1a[l], w1c[l], upd_W2[l], b2[l])
            else:
                e[h], msg[h] = _tc_edge(xs, pd, e[h], w1a[l], w1c[l],
                                        upd_W2[l], b2[l])
            aggr[h] = _sc_aggr(msg[h], dstm[h], zeros)
        if l < DEPTH - 1:
            x, xp, p2 = _tc_node(x, aggr[0], aggr[1], conv_W1[l], cb1[l],
                                 conv_W2[l], cb2[l], gam[l], bet[l],
                                 w1b[l + 1], nb1[l + 1])
        else:
            out = _tc_last(x, aggr[0], aggr[1], conv_W1[l], cb1[l],
                           conv_W2[l], cb2[l], gam[l], bet[l], batch_row,
                           ro_W, ro_b.reshape(1, H))
    return out
